# baseline (speedup 1.0000x reference)
import functools

import jax
import jax.numpy as jnp
from jax import lax
from jax.experimental import pallas as pl
from jax.experimental.pallas import tpu as pltpu
from jax.experimental.pallas import tpu_sc as plsc

N_NODE = 10000
EMB = 128
N_EDGES = 320000
LAYERS = 3

E = 128
PK = 2 * E
NCHUNKS = N_EDGES // E
NW = 32
KU = NCHUNKS // NW
NEXTRA = NCHUNKS - KU * NW
ROWCH = 80
NROWCH = N_NODE // ROWCH
KROW = (NROWCH + 15) // 16

_mesh = plsc.VectorSubcoreMesh(core_axis_name="c", subcore_axis_name="s")


@functools.partial(
    pl.kernel,
    out_type=[
        jax.ShapeDtypeStruct((N_NODE, EMB), jnp.float32),
        jax.ShapeDtypeStruct((N_NODE, EMB), jnp.float32),
    ],
    mesh=_mesh,
    scratch_types=[
        pltpu.VMEM_SHARED((N_NODE, EMB), jnp.float32),
        pltpu.VMEM((PK,), jnp.int32),
        pltpu.VMEM((PK,), jnp.int32),
        pltpu.VMEM((E,), jnp.float32),
        pltpu.VMEM((E,), jnp.float32),
        pltpu.VMEM((E,), jnp.int32),
        pltpu.VMEM((E,), jnp.int32),
        pltpu.VMEM((E, EMB), jnp.float32),
        pltpu.VMEM((E, EMB), jnp.float32),
        pltpu.SemaphoreType.DMA,
        pltpu.SemaphoreType.DMA,
        pltpu.SemaphoreType.DMA,
        pltpu.SemaphoreType.DMA,
        pltpu.SemaphoreType.DMA,
        pltpu.SemaphoreType.DMA,
    ],
)
def _spmm(x_hbm, edata_hbm, w_hbm, p0_hbm, p1_hbm,
          acc_sh, eb0, eb1, wb0, wb1, db0, db1, rw0, rw1,
          isem0, isem1, gsem0, gsem1, ssem0, ssem1):
    cid = lax.axis_index("c")
    sid = lax.axis_index("s")
    wid = sid * 2 + cid

    eb = [eb0, eb1]
    wb = [wb0, wb1]
    db = [db0, db1]
    rw = [rw0, rw1]
    isem = [isem0, isem1]
    gsem = [gsem0, gsem1]
    ssem = [ssem0, ssem1]

    def _zrow(r, carry):
        for i in range(8):
            rw0[r, pl.ds(i * 16, 16)] = jnp.zeros((16,), jnp.float32)
        return carry
    lax.fori_loop(0, ROWCH, _zrow, 0)

    for k in range(KROW):
        c = k * 16 + sid

        @pl.when(c < NROWCH)
        def _():
            pltpu.sync_copy(rw0.at[pl.ds(0, ROWCH)],
                            acc_sh.at[pl.ds(c * ROWCH, ROWCH)])
    plsc.subcore_barrier()

    def start_idx(c, p):
        j = wid + c * NW
        pltpu.async_copy(edata_hbm.at[pl.ds(j * PK, PK)], eb[p], isem[p])
        pltpu.async_copy(w_hbm.at[pl.ds(j * E, E)], wb[p], isem[p])

    def wait_idx(p):
        pltpu.make_async_copy(edata_hbm.at[pl.ds(0, PK)], eb[p], isem[p]).wait()
        pltpu.make_async_copy(w_hbm.at[pl.ds(0, E)], wb[p], isem[p]).wait()

    def start_gather(p):
        pltpu.async_copy(x_hbm.at[eb[p].at[pl.ds(0, E)]], rw[p], gsem[p])

    def wait_gather(p):
        pltpu.make_async_copy(x_hbm.at[eb[p].at[pl.ds(0, E)]],
                              rw[p], gsem[p]).wait()

    def start_scatter(p):
        pltpu.async_copy(rw[p], acc_sh.at[db[p]], ssem[p], add=True)

    def wait_scatter(p):
        pltpu.make_async_copy(rw[p], acc_sh.at[db[p]], ssem[p]).wait()

    def scale_and_dst(p):
        ebp, wbp, rwp, dbp = eb[p], wb[p], rw[p], db[p]

        def _scale(g, carry):
            wvec = wbp[pl.ds(g * 16, 16)]
            for rr in range(16):
                r = g * 16 + rr
                w = wvec[rr]
                for i in range(8):
                    sl = pl.ds(i * 16, 16)
                    rwp[r, sl] = rwp[r, sl] * w
            return carry
        lax.fori_loop(0, E // 16, _scale, 0)
        for i in range(8):
            sl = pl.ds(i * 16, 16)
            dbp[sl] = ebp[pl.ds(E + i * 16, 16)]

    def proc(c, p, gather_next, idx_next, wait_prev_scatter):
        q = 1 - p
        wait_gather(p)
        if gather_next:
            if wait_prev_scatter:
                wait_scatter(q)
            wait_idx(q)
            start_gather(q)
        scale_and_dst(p)
        start_scatter(p)
        if idx_next:
            start_idx(c + 2, p)

    start_idx(0, 0)
    start_idx(1, 1)
    wait_idx(0)
    start_gather(0)
    proc(0, 0, True, True, False)

    def _pair(k, carry):
        c = 2 * k + 1
        proc(c, 1, True, True, True)
        proc(c + 1, 0, True, True, True)
        return carry
    lax.fori_loop(0, (KU - 4) // 2, _pair, 0)

    proc(KU - 3, 1, True, True, True)
    proc(KU - 2, 0, True, False, True)
    proc(KU - 1, 1, False, False, False)
    wait_scatter(0)
    wait_scatter(1)

    @pl.when(wid < NEXTRA)
    def _():
        j = KU * NW + wid
        pltpu.sync_copy(edata_hbm.at[pl.ds(j * PK, PK)], eb0)
        pltpu.sync_copy(w_hbm.at[pl.ds(j * E, E)], wb0)
        pltpu.async_copy(x_hbm.at[eb0.at[pl.ds(0, E)]], rw0, gsem0).wait()
        scale_and_dst(0)
        pltpu.sync_copy(rw0, acc_sh.at[db0], add=True)

    plsc.subcore_barrier()

    for k in range(KROW):
        c = k * 16 + sid
        sl = pl.ds(c * ROWCH, ROWCH)

        @pl.when(jnp.logical_and(c < NROWCH, cid == 0))
        def _():
            pltpu.sync_copy(acc_sh.at[sl], p0_hbm.at[sl])

        @pl.when(jnp.logical_and(c < NROWCH, cid == 1))
        def _():
            pltpu.sync_copy(acc_sh.at[sl], p1_hbm.at[sl])


def _merge_body(scale, p0_ref, p1_ref, acc_ref, x_ref, accout_ref):
    s = p0_ref[...] + p1_ref[...]
    x_ref[...] = s
    accout_ref[...] = (acc_ref[...] + s) * scale


def _merge(p0, p1, acc, scale):
    grid = 10
    rows = N_NODE // grid
    spec = pl.BlockSpec((rows, EMB), lambda i: (i, 0))
    return pl.pallas_call(
        functools.partial(_merge_body, scale),
        grid=(grid,),
        in_specs=[spec, spec, spec],
        out_specs=[spec, spec],
        out_shape=[
            jax.ShapeDtypeStruct((N_NODE, EMB), jnp.float32),
            jax.ShapeDtypeStruct((N_NODE, EMB), jnp.float32),
        ],
    )(p0, p1, acc)


@jax.jit
def kernel(embedding, edge_index, edge_weight):
    src = edge_index[1].reshape(NCHUNKS, E)
    dst = edge_index[0].reshape(NCHUNKS, E)
    edata = jnp.stack([src, dst], axis=1).reshape(NCHUNKS * PK)
    x = embedding
    acc = embedding
    for layer in range(LAYERS):
        p0, p1 = _spmm(x, edata, edge_weight)
        scale = (1.0 / (LAYERS + 1)) if layer == LAYERS - 1 else 1.0
        x, acc = _merge(p0, p1, acc, scale)
    return acc


# speedup vs baseline: 1.0699x; 1.0699x over previous
import functools

import jax
import jax.numpy as jnp
from jax import lax
from jax.experimental import pallas as pl
from jax.experimental.pallas import tpu as pltpu
from jax.experimental.pallas import tpu_sc as plsc

N_NODE = 10000
EMB = 128
N_EDGES = 320000
LAYERS = 3

E = 80
PK = 2 * E
NCHUNKS = N_EDGES // E
NW = 32
KU = NCHUNKS // NW
NEXTRA = NCHUNKS - KU * NW
ROWCH = 80
NROWCH = N_NODE // ROWCH
KROW = (NROWCH + 15) // 16

_mesh = plsc.VectorSubcoreMesh(core_axis_name="c", subcore_axis_name="s")


@functools.partial(
    pl.kernel,
    out_type=[
        jax.ShapeDtypeStruct((N_NODE, EMB), jnp.float32),
        jax.ShapeDtypeStruct((N_NODE, EMB), jnp.float32),
    ],
    mesh=_mesh,
    scratch_types=[
        pltpu.VMEM_SHARED((N_NODE, EMB), jnp.float32),
        pltpu.VMEM((PK,), jnp.int32),
        pltpu.VMEM((PK,), jnp.int32),
        pltpu.VMEM((PK,), jnp.int32),
        pltpu.VMEM((PK,), jnp.int32),
        pltpu.VMEM((E,), jnp.float32),
        pltpu.VMEM((E,), jnp.float32),
        pltpu.VMEM((E,), jnp.float32),
        pltpu.VMEM((E,), jnp.float32),
        pltpu.VMEM((E,), jnp.int32),
        pltpu.VMEM((E,), jnp.int32),
        pltpu.VMEM((E,), jnp.int32),
        pltpu.VMEM((E,), jnp.int32),
        pltpu.VMEM((E, EMB), jnp.float32),
        pltpu.VMEM((E, EMB), jnp.float32),
        pltpu.VMEM((E, EMB), jnp.float32),
        pltpu.VMEM((E, EMB), jnp.float32),
        pltpu.SemaphoreType.DMA,
        pltpu.SemaphoreType.DMA,
        pltpu.SemaphoreType.DMA,
        pltpu.SemaphoreType.DMA,
        pltpu.SemaphoreType.DMA,
        pltpu.SemaphoreType.DMA,
        pltpu.SemaphoreType.DMA,
        pltpu.SemaphoreType.DMA,
        pltpu.SemaphoreType.DMA,
        pltpu.SemaphoreType.DMA,
        pltpu.SemaphoreType.DMA,
        pltpu.SemaphoreType.DMA,
    ],
)
def _spmm(x_hbm, edata_hbm, w_hbm, p0_hbm, p1_hbm,
          acc_sh, eb0, eb1, eb2, eb3, wb0, wb1, wb2, wb3,
          db0, db1, db2, db3, rw0, rw1, rw2, rw3,
          isem0, isem1, isem2, isem3, gsem0, gsem1, gsem2, gsem3,
          ssem0, ssem1, ssem2, ssem3):
    cid = lax.axis_index("c")
    sid = lax.axis_index("s")
    wid = sid * 2 + cid

    eb = [eb0, eb1, eb2, eb3]
    wb = [wb0, wb1, wb2, wb3]
    db = [db0, db1, db2, db3]
    rw = [rw0, rw1, rw2, rw3]
    isem = [isem0, isem1, isem2, isem3]
    gsem = [gsem0, gsem1, gsem2, gsem3]
    ssem = [ssem0, ssem1, ssem2, ssem3]

    def _zrow(r, carry):
        for i in range(8):
            rw0[r, pl.ds(i * 16, 16)] = jnp.zeros((16,), jnp.float32)
        return carry
    lax.fori_loop(0, ROWCH, _zrow, 0)

    for k in range(KROW):
        c = k * 16 + sid

        @pl.when(c < NROWCH)
        def _():
            pltpu.sync_copy(rw0.at[pl.ds(0, ROWCH)],
                            acc_sh.at[pl.ds(c * ROWCH, ROWCH)])
    plsc.subcore_barrier()

    def start_idx(c, p):
        j = wid + c * NW
        pltpu.async_copy(edata_hbm.at[pl.ds(j * PK, PK)], eb[p], isem[p])
        pltpu.async_copy(w_hbm.at[pl.ds(j * E, E)], wb[p], isem[p])

    def wait_idx(p):
        pltpu.make_async_copy(edata_hbm.at[pl.ds(0, PK)], eb[p], isem[p]).wait()
        pltpu.make_async_copy(w_hbm.at[pl.ds(0, E)], wb[p], isem[p]).wait()

    def start_gather(p):
        pltpu.async_copy(x_hbm.at[eb[p].at[pl.ds(0, E)]], rw[p], gsem[p])

    def wait_gather(p):
        pltpu.make_async_copy(x_hbm.at[eb[p].at[pl.ds(0, E)]],
                              rw[p], gsem[p]).wait()

    def start_scatter(p):
        pltpu.async_copy(rw[p], acc_sh.at[db[p]], ssem[p], add=True)

    def wait_scatter(p):
        pltpu.make_async_copy(rw[p], acc_sh.at[db[p]], ssem[p]).wait()

    def scale_and_dst(p):
        ebp, wbp, rwp, dbp = eb[p], wb[p], rw[p], db[p]

        def _scale(g, carry):
            wvec = wbp[pl.ds(g * 16, 16)]
            for rr in range(16):
                r = g * 16 + rr
                w = wvec[rr]
                for i in range(8):
                    sl = pl.ds(i * 16, 16)
                    rwp[r, sl] = rwp[r, sl] * w
            return carry
        lax.fori_loop(0, E // 16, _scale, 0)
        for i in range(E // 16):
            sl = pl.ds(i * 16, 16)
            dbp[sl] = ebp[pl.ds(E + i * 16, 16)]

    def proc(c, p, gather_ahead, idx_ahead, wait_prev_scatter):
        p2 = (p + 2) % 4
        wait_gather(p)
        if gather_ahead:
            if wait_prev_scatter:
                wait_scatter(p2)
            wait_idx(p2)
            start_gather(p2)
        scale_and_dst(p)
        start_scatter(p)
        if idx_ahead:
            start_idx(c + 4, p)

    for c0 in range(4):
        start_idx(c0, c0)
    wait_idx(0)
    start_gather(0)
    wait_idx(1)
    start_gather(1)
    proc(0, 0, True, True, False)
    proc(1, 1, True, True, False)

    nq = (KU - 6) // 4
    steady_end = 2 + 4 * nq

    def _quad(k, carry):
        c = 4 * k + 2
        proc(c, 2, True, True, True)
        proc(c + 1, 3, True, True, True)
        proc(c + 2, 0, True, True, True)
        proc(c + 3, 1, True, True, True)
        return carry
    lax.fori_loop(0, nq, _quad, 0)

    for c in range(steady_end, KU - 4):
        proc(c, c % 4, True, True, True)
    for c in range(KU - 4, KU - 2):
        proc(c, c % 4, True, False, True)
    for c in range(KU - 2, KU):
        proc(c, c % 4, False, False, False)
    for c in range(KU - 4, KU):
        wait_scatter(c % 4)

    plsc.subcore_barrier()

    for k in range(KROW):
        c = k * 16 + sid
        sl = pl.ds(c * ROWCH, ROWCH)

        @pl.when(jnp.logical_and(c < NROWCH, cid == 0))
        def _():
            pltpu.sync_copy(acc_sh.at[sl], p0_hbm.at[sl])

        @pl.when(jnp.logical_and(c < NROWCH, cid == 1))
        def _():
            pltpu.sync_copy(acc_sh.at[sl], p1_hbm.at[sl])


def _merge_body(scale, p0_ref, p1_ref, acc_ref, x_ref, accout_ref):
    s = p0_ref[...] + p1_ref[...]
    x_ref[...] = s
    accout_ref[...] = (acc_ref[...] + s) * scale


def _merge(p0, p1, acc, scale):
    grid = 10
    rows = N_NODE // grid
    spec = pl.BlockSpec((rows, EMB), lambda i: (i, 0))
    return pl.pallas_call(
        functools.partial(_merge_body, scale),
        grid=(grid,),
        in_specs=[spec, spec, spec],
        out_specs=[spec, spec],
        out_shape=[
            jax.ShapeDtypeStruct((N_NODE, EMB), jnp.float32),
            jax.ShapeDtypeStruct((N_NODE, EMB), jnp.float32),
        ],
    )(p0, p1, acc)


@jax.jit
def kernel(embedding, edge_index, edge_weight):
    src = edge_index[1].reshape(NCHUNKS, E)
    dst = edge_index[0].reshape(NCHUNKS, E)
    edata = jnp.stack([src, dst], axis=1).reshape(NCHUNKS * PK)
    x = embedding
    acc = embedding
    for layer in range(LAYERS):
        p0, p1 = _spmm(x, edata, edge_weight)
        scale = (1.0 / (LAYERS + 1)) if layer == LAYERS - 1 else 1.0
        x, acc = _merge(p0, p1, acc, scale)
    return acc


# speedup vs baseline: 1.1000x; 1.0281x over previous
import functools

import jax
import jax.numpy as jnp
from jax import lax
from jax.experimental import pallas as pl
from jax.experimental.pallas import tpu as pltpu
from jax.experimental.pallas import tpu_sc as plsc

N_NODE = 10000
EMB = 128
N_EDGES = 320000
LAYERS = 3

E = 80
PK = 2 * E
NCHUNKS = N_EDGES // E
NW = 32
KU = NCHUNKS // NW
NEXTRA = NCHUNKS - KU * NW
ROWCH = 80
NROWCH = N_NODE // ROWCH
KROW = (NROWCH + 15) // 16

_mesh = plsc.VectorSubcoreMesh(core_axis_name="c", subcore_axis_name="s")


@functools.partial(
    pl.kernel,
    out_type=[
        jax.ShapeDtypeStruct((N_NODE, EMB), jnp.float32),
        jax.ShapeDtypeStruct((N_NODE, EMB), jnp.float32),
    ],
    mesh=_mesh,
    scratch_types=[
        pltpu.VMEM_SHARED((N_NODE, EMB), jnp.float32),
        pltpu.VMEM((PK,), jnp.int32),
        pltpu.VMEM((PK,), jnp.int32),
        pltpu.VMEM((PK,), jnp.int32),
        pltpu.VMEM((PK,), jnp.int32),
        pltpu.VMEM((E,), jnp.float32),
        pltpu.VMEM((E,), jnp.float32),
        pltpu.VMEM((E,), jnp.float32),
        pltpu.VMEM((E,), jnp.float32),
        pltpu.VMEM((E,), jnp.int32),
        pltpu.VMEM((E,), jnp.int32),
        pltpu.VMEM((E,), jnp.int32),
        pltpu.VMEM((E,), jnp.int32),
        pltpu.VMEM((E, EMB), jnp.float32),
        pltpu.VMEM((E, EMB), jnp.float32),
        pltpu.VMEM((E, EMB), jnp.float32),
        pltpu.VMEM((E, EMB), jnp.float32),
        pltpu.SemaphoreType.DMA,
        pltpu.SemaphoreType.DMA,
        pltpu.SemaphoreType.DMA,
        pltpu.SemaphoreType.DMA,
        pltpu.SemaphoreType.DMA,
        pltpu.SemaphoreType.DMA,
        pltpu.SemaphoreType.DMA,
        pltpu.SemaphoreType.DMA,
        pltpu.SemaphoreType.DMA,
        pltpu.SemaphoreType.DMA,
        pltpu.SemaphoreType.DMA,
        pltpu.SemaphoreType.DMA,
    ],
)
def _spmm(x_hbm, edata_hbm, w_hbm, p0_hbm, p1_hbm,
          acc_sh, eb0, eb1, eb2, eb3, wb0, wb1, wb2, wb3,
          db0, db1, db2, db3, rw0, rw1, rw2, rw3,
          isem0, isem1, isem2, isem3, gsem0, gsem1, gsem2, gsem3,
          ssem0, ssem1, ssem2, ssem3):
    cid = lax.axis_index("c")
    sid = lax.axis_index("s")
    wid = sid * 2 + cid

    eb = [eb0, eb1, eb2, eb3]
    wb = [wb0, wb1, wb2, wb3]
    db = [db0, db1, db2, db3]
    rw = [rw0, rw1, rw2, rw3]
    isem = [isem0, isem1, isem2, isem3]
    gsem = [gsem0, gsem1, gsem2, gsem3]
    ssem = [ssem0, ssem1, ssem2, ssem3]

    def _zrow(r, carry):
        for i in range(8):
            rw0[r, pl.ds(i * 16, 16)] = jnp.zeros((16,), jnp.float32)
        return carry
    lax.fori_loop(0, ROWCH, _zrow, 0)

    for k in range(KROW):
        c = k * 16 + sid

        @pl.when(c < NROWCH)
        def _():
            pltpu.sync_copy(rw0.at[pl.ds(0, ROWCH)],
                            acc_sh.at[pl.ds(c * ROWCH, ROWCH)])
    plsc.subcore_barrier()

    def start_idx(c, p):
        j = wid + c * NW
        pltpu.async_copy(edata_hbm.at[pl.ds(j * PK, PK)], eb[p], isem[p])
        pltpu.async_copy(w_hbm.at[pl.ds(j * E, E)], wb[p], isem[p])

    def wait_idx(p):
        pltpu.make_async_copy(edata_hbm.at[pl.ds(0, PK)], eb[p], isem[p]).wait()
        pltpu.make_async_copy(w_hbm.at[pl.ds(0, E)], wb[p], isem[p]).wait()

    def start_gather(p):
        pltpu.async_copy(x_hbm.at[eb[p].at[pl.ds(0, E)]], rw[p], gsem[p])

    def wait_gather(p):
        pltpu.make_async_copy(x_hbm.at[eb[p].at[pl.ds(0, E)]],
                              rw[p], gsem[p]).wait()

    def start_scatter(p):
        pltpu.async_copy(rw[p], acc_sh.at[db[p]], ssem[p], add=True)

    def wait_scatter(p):
        pltpu.make_async_copy(rw[p], acc_sh.at[db[p]], ssem[p]).wait()

    def scale_and_dst(p):
        ebp, wbp, rwp, dbp = eb[p], wb[p], rw[p], db[p]

        def _scale(g, carry):
            wvec = wbp[pl.ds(g * 16, 16)]
            for rr in range(16):
                r = g * 16 + rr
                w = wvec[rr]
                for i in range(8):
                    sl = pl.ds(i * 16, 16)
                    rwp[r, sl] = rwp[r, sl] * w
            return carry
        lax.fori_loop(0, E // 16, _scale, 0)
        for i in range(E // 16):
            sl = pl.ds(i * 16, 16)
            dbp[sl] = ebp[pl.ds(E + i * 16, 16)]

    def proc(c, p, gather_ahead, idx_ahead, wait_prev_scatter):
        p2 = (p + 2) % 4
        wait_gather(p)
        if gather_ahead:
            if wait_prev_scatter:
                wait_scatter(p2)
            wait_idx(p2)
            start_gather(p2)
        scale_and_dst(p)
        start_scatter(p)
        if idx_ahead:
            start_idx(c + 4, p)

    for c0 in range(4):
        start_idx(c0, c0)
    wait_idx(0)
    start_gather(0)
    wait_idx(1)
    start_gather(1)
    proc(0, 0, True, True, False)
    proc(1, 1, True, True, False)

    nq = (KU - 6) // 4
    steady_end = 2 + 4 * nq

    def _quad(k, carry):
        c = 4 * k + 2
        proc(c, 2, True, True, True)
        proc(c + 1, 3, True, True, True)
        proc(c + 2, 0, True, True, True)
        proc(c + 3, 1, True, True, True)
        return carry
    lax.fori_loop(0, nq, _quad, 0)

    for c in range(steady_end, KU - 4):
        proc(c, c % 4, True, True, True)
    for c in range(KU - 4, KU - 2):
        proc(c, c % 4, True, False, True)
    for c in range(KU - 2, KU):
        proc(c, c % 4, False, False, False)
    for c in range(KU - 4, KU):
        wait_scatter(c % 4)

    plsc.subcore_barrier()

    for k in range(KROW):
        c = k * 16 + sid
        sl = pl.ds(c * ROWCH, ROWCH)

        @pl.when(jnp.logical_and(c < NROWCH, cid == 0))
        def _():
            pltpu.sync_copy(acc_sh.at[sl], p0_hbm.at[sl])

        @pl.when(jnp.logical_and(c < NROWCH, cid == 1))
        def _():
            pltpu.sync_copy(acc_sh.at[sl], p1_hbm.at[sl])


def _merge_body(scale, p0_ref, p1_ref, acc_ref, x_ref, accout_ref):
    s = p0_ref[...] + p1_ref[...]
    x_ref[...] = s
    accout_ref[...] = (acc_ref[...] + s) * scale


def _merge(p0, p1, acc, scale):
    grid = 10
    rows = N_NODE // grid
    spec = pl.BlockSpec((rows, EMB), lambda i: (i, 0))
    return pl.pallas_call(
        functools.partial(_merge_body, scale),
        grid=(grid,),
        in_specs=[spec, spec, spec],
        out_specs=[spec, spec],
        out_shape=[
            jax.ShapeDtypeStruct((N_NODE, EMB), jnp.float32),
            jax.ShapeDtypeStruct((N_NODE, EMB), jnp.float32),
        ],
    )(p0, p1, acc)


@jax.jit
def kernel(embedding, edge_index, edge_weight):
    src = edge_index[1].reshape(NCHUNKS, E)
    dst = edge_index[0].reshape(NCHUNKS, E)
    edata = jnp.stack([src, dst], axis=1).reshape(NCHUNKS * PK)
    x = embedding
    acc = embedding
    for layer in range(LAYERS):
        p0, p1 = _spmm(x, edata, edge_weight)
        scale = (1.0 / (LAYERS + 1)) if layer == LAYERS - 1 else 1.0
        x = p0 + p1
        acc = (acc + x) * scale
    return acc
